# baseline (speedup 1.0000x reference)
import functools

import jax
import jax.numpy as jnp
from jax import lax
from jax.experimental import pallas as pl
from jax.experimental.pallas import tpu as pltpu
from jax.experimental.pallas import tpu_sc as plsc

B = 4096
S = 200
D = 32
NUM_ROWS = 100000

NC = 2
NS = 16
NW = NC * NS
RPW = B // NW
CB = 8
CHUNKS = RPW // CB
SP = 208
NIDX = CB * SP // 128
NVEC = CB * SP // 16


def _body(x_hbm, m_hbm, w_hbm, out_hbm,
          idx_v, msk_v, idx2_v, rows_v, w0_v, outst_v, sem):
    wid = lax.axis_index("s") * NC + lax.axis_index("c")
    base = wid * RPW

    pltpu.sync_copy(w_hbm.at[pl.ds(0, 1)], w0_v)
    w0a = w0_v[0, pl.ds(0, 16)]
    w0b = w0_v[0, pl.ds(16, 16)]

    zero16 = jnp.zeros((16,), jnp.int32)
    for t in range(NVEC):
        msk_v[pl.ds(t * 16, 16)] = zero16

    inv_s = jnp.float32(1.0 / S)

    def chunk_body(g, carry):
        row0 = base + g * CB
        cps = []
        for j in range(CB):
            cps.append(pltpu.async_copy(
                x_hbm.at[row0 + j], idx_v.at[pl.ds(j * SP, S)], sem))
            cps.append(pltpu.async_copy(
                m_hbm.at[row0 + j], msk_v.at[pl.ds(j * SP, S)], sem))
        for cp in cps:
            cp.wait()

        for t in range(NVEC):
            m16 = msk_v[pl.ds(t * 16, 16)]
            i16 = idx_v[pl.ds(t * 16, 16)]
            idx2_v[t // 8, pl.ds((t % 8) * 16, 16)] = i16 * m16

        gcs = [pltpu.async_copy(w_hbm.at[idx2_v.at[k]],
                                rows_v.at[pl.ds(k * 128, 128)], sem)
               for k in range(NIDX)]
        for gc in gcs:
            gc.wait()

        for j in range(CB):
            msum = None
            for u in range(SP // 16):
                mb = msk_v[pl.ds(j * SP + u * 16, 16)] != 0
                p = plsc.all_reduce_population_count(mb)
                msum = p if msum is None else msum + p
            n0f = (SP - msum).astype(jnp.float32)

            def srow(s, accs):
                a0, a1 = accs
                r = j * SP + s
                return (a0 + rows_v[r, pl.ds(0, 16)],
                        a1 + rows_v[r, pl.ds(16, 16)])

            a0, a1 = lax.fori_loop(
                0, SP, srow,
                (jnp.zeros((16,), jnp.float32), jnp.zeros((16,), jnp.float32)))
            outst_v[j, pl.ds(0, 16)] = (a0 - n0f * w0a) * inv_s
            outst_v[j, pl.ds(16, 16)] = (a1 - n0f * w0b) * inv_s

        pltpu.sync_copy(outst_v, out_hbm.at[pl.ds(row0, CB)])
        return carry

    lax.fori_loop(0, CHUNKS, chunk_body, 0)


@jax.jit
def _run(x, mask_i, w):
    mesh = plsc.VectorSubcoreMesh(core_axis_name="c", subcore_axis_name="s")
    f = pl.kernel(
        _body,
        out_type=jax.ShapeDtypeStruct((B, D), jnp.float32),
        mesh=mesh,
        compiler_params=pltpu.CompilerParams(
            needs_layout_passes=False, use_tc_tiling_on_sc=False),
        scratch_types=[
            pltpu.VMEM((CB * SP,), jnp.int32),
            pltpu.VMEM((CB * SP,), jnp.int32),
            pltpu.VMEM((NIDX, 128), jnp.int32),
            pltpu.VMEM((CB * SP, D), jnp.float32),
            pltpu.VMEM((1, D), jnp.float32),
            pltpu.VMEM((CB, D), jnp.float32),
            pltpu.SemaphoreType.DMA,
        ],
    )
    return f(x, mask_i, w)


def kernel(x, mask, W):
    return _run(x, mask.astype(jnp.int32), W)


# speedup vs baseline: 7.0746x; 7.0746x over previous
import functools

import jax
import jax.numpy as jnp
from jax import lax
from jax.experimental import pallas as pl
from jax.experimental.pallas import tpu as pltpu
from jax.experimental.pallas import tpu_sc as plsc

B = 4096
S = 200
D = 32
NUM_ROWS = 100000

NC = 2
NS = 16
HALF = NUM_ROWS // NC
RPT = B // NS
CB = 4
CHUNKS = RPT // CB
SP = 224
NE = CB * SP
NIDX = NE // 128
NVEC = SP // 16


def _body(x_hbm, m_hbm, w_hbm, out_hbm,
          idx_v, msk_v, rows_v, w0_v, outst_v, shared_w, sem):
    c = lax.axis_index("c")
    sid = lax.axis_index("s")
    base = sid * RPT
    hbase = c * HALF

    @pl.when(sid == 0)
    def _stage():
        pltpu.sync_copy(w_hbm.at[pl.ds(hbase, HALF)], shared_w)

    plsc.subcore_barrier()
    pltpu.sync_copy(shared_w.at[pl.ds(0, 1)], w0_v)
    w0a = w0_v[0, pl.ds(0, 16)]
    w0b = w0_v[0, pl.ds(16, 16)]

    zero16 = jnp.zeros((16,), jnp.int32)
    for j in range(CB):
        for u in range(NVEC):
            msk_v[j, pl.ds(u * 16, 16)] = zero16

    inv_s = jnp.float32(1.0 / S)

    def chunk_body(g, carry):
        row0 = base + g * CB
        cps = []
        for j in range(CB):
            cps.append(pltpu.async_copy(
                x_hbm.at[row0 + j], idx_v.at[pl.ds(j * SP, S)], sem))
            cps.append(pltpu.async_copy(
                m_hbm.at[row0 + j], msk_v.at[j, pl.ds(0, S)], sem))
        for cp in cps:
            cp.wait()

        for j in range(CB):
            for u in range(NVEC):
                e = j * SP + u * 16
                m16 = msk_v[j, pl.ds(u * 16, 16)]
                i16 = idx_v[pl.ds(e, 16)] - hbase
                keep = ((m16 != 0) & (i16 >= 0)
                        & (i16 < HALF)).astype(jnp.int32)
                idx_v[pl.ds(e, 16)] = i16 * keep
                msk_v[j, pl.ds(u * 16, 16)] = keep

        for k in range(NIDX):
            pltpu.async_copy(shared_w.at[idx_v.at[pl.ds(k * 128, 128)]],
                             rows_v.at[pl.ds(k * 128, 128)], sem).wait()

        for j in range(CB):
            msum = None
            for u in range(NVEC):
                mb = msk_v[j, pl.ds(u * 16, 16)] != 0
                p = plsc.all_reduce_population_count(mb)
                msum = p if msum is None else msum + p
            n0f = (SP - msum).astype(jnp.float32)

            def srow(s, accs):
                a0, a1 = accs
                r = j * SP + s
                return (a0 + rows_v[r, pl.ds(0, 16)],
                        a1 + rows_v[r, pl.ds(16, 16)])

            a0, a1 = lax.fori_loop(
                0, SP, srow,
                (jnp.zeros((16,), jnp.float32), jnp.zeros((16,), jnp.float32)))
            outst_v[j, pl.ds(0, 16)] = (a0 - n0f * w0a) * inv_s
            outst_v[j, pl.ds(16, 16)] = (a1 - n0f * w0b) * inv_s

        pltpu.sync_copy(outst_v, out_hbm.at[c, pl.ds(row0, CB)])
        return carry

    lax.fori_loop(0, CHUNKS, chunk_body, 0)


def _combine_body(p_ref, o_ref):
    o_ref[...] = p_ref[0] + p_ref[1]


@jax.jit
def _run(x, mask_i, w):
    mesh = plsc.VectorSubcoreMesh(core_axis_name="c", subcore_axis_name="s")
    f = pl.kernel(
        _body,
        out_type=jax.ShapeDtypeStruct((NC, B, D), jnp.float32),
        mesh=mesh,
        compiler_params=pltpu.CompilerParams(
            needs_layout_passes=False, use_tc_tiling_on_sc=False),
        scratch_types=[
            pltpu.VMEM((NE,), jnp.int32),
            pltpu.VMEM((CB, SP), jnp.int32),
            pltpu.VMEM((NE, D), jnp.float32),
            pltpu.VMEM((1, D), jnp.float32),
            pltpu.VMEM((CB, D), jnp.float32),
            pltpu.VMEM_SHARED((HALF, D), jnp.float32),
            pltpu.SemaphoreType.DMA,
        ],
    )
    partial = f(x, mask_i, w)
    return pl.pallas_call(
        _combine_body,
        out_shape=jax.ShapeDtypeStruct((B, D), jnp.float32),
    )(partial)


def kernel(x, mask, W):
    return _run(x, mask.astype(jnp.int32), W)


# speedup vs baseline: 15.5862x; 2.2031x over previous
import functools

import jax
import jax.numpy as jnp
from jax import lax
from jax.experimental import pallas as pl
from jax.experimental.pallas import tpu as pltpu
from jax.experimental.pallas import tpu_sc as plsc

B = 4096
S = 200
D = 32
NUM_ROWS = 100000

NC = 2
NS = 16
HALF = NUM_ROWS // NC
RPT = B // NS
CB = 4
CHUNKS = RPT // CB
NEC = CB * S
NGRP = NEC // 16
IDXC = 1024
NROW = 896


def _body(x_hbm, m_hbm, w_hbm, out_hbm,
          idx_c, msk_v, rows_v, outst_v, shared_w, sem):
    c = lax.axis_index("c")
    sid = lax.axis_index("s")
    base = sid * RPT
    hbase = c * HALF

    @pl.when(sid == 0)
    def _stage():
        pltpu.sync_copy(w_hbm.at[pl.ds(hbase, HALF)], shared_w)

    plsc.subcore_barrier()

    inv_s = jnp.float32(1.0 / S)
    zero16i = jnp.zeros((16,), jnp.int32)
    zero16f = jnp.zeros((16,), jnp.float32)
    lo8 = lax.iota(jnp.int32, 16) < 8

    def chunk_body(g, carry):
        row0 = base + g * CB
        e0 = row0 * S
        cp1 = pltpu.async_copy(
            x_hbm.at[pl.ds(e0, NEC)], idx_c.at[pl.ds(0, NEC)], sem)
        cp2 = pltpu.async_copy(
            m_hbm.at[pl.ds(e0, NEC)], msk_v.at[pl.ds(0, NEC)], sem)
        cp1.wait()
        cp2.wait()

        off = jnp.int32(0)
        bounds = [off]
        for grp in range(NGRP):
            e = grp * 16
            m16 = msk_v[pl.ds(e, 16)]
            i16 = idx_c[pl.ds(e, 16)] - hbase
            keep = (m16 != 0) & (i16 >= 0) & (i16 < HALF)
            if grp in (12, 37):
                ka = keep & lo8
                plsc.store_compressed(idx_c.at[pl.ds(off, 16)], i16, mask=ka)
                off = off + plsc.all_reduce_population_count(ka)[0]
                bounds.append(off)
                kb = keep & (~lo8)
                plsc.store_compressed(idx_c.at[pl.ds(off, 16)], i16, mask=kb)
                off = off + plsc.all_reduce_population_count(kb)[0]
            else:
                plsc.store_compressed(idx_c.at[pl.ds(off, 16)], i16,
                                      mask=keep)
                off = off + plsc.all_reduce_population_count(keep)[0]
                if grp == 24:
                    bounds.append(off)
        bounds.append(off)

        for t in range(8):
            idx_c[pl.ds(off + t * 16, 16)] = zero16i

        ngrp = (off + 127) // 128

        def gbody(k, carry2):
            pltpu.async_copy(shared_w.at[idx_c.at[pl.ds(k * 128, 128)]],
                             rows_v.at[pl.ds(k * 128, 128)], sem).wait()
            return carry2

        lax.fori_loop(0, ngrp, gbody, 0)

        for j in range(CB):
            def srow(t, accs):
                a0, a1 = accs
                return (a0 + rows_v[t, pl.ds(0, 16)],
                        a1 + rows_v[t, pl.ds(16, 16)])

            a0, a1 = lax.fori_loop(bounds[j], bounds[j + 1], srow,
                                   (zero16f, zero16f))
            outst_v[j, pl.ds(0, 16)] = a0 * inv_s
            outst_v[j, pl.ds(16, 16)] = a1 * inv_s

        pltpu.sync_copy(outst_v, out_hbm.at[c, pl.ds(row0, CB)])
        return carry

    lax.fori_loop(0, CHUNKS, chunk_body, 0)


def _combine_body(p_ref, o_ref):
    o_ref[...] = p_ref[0] + p_ref[1]


@jax.jit
def _run(x, mask_i, w):
    mesh = plsc.VectorSubcoreMesh(core_axis_name="c", subcore_axis_name="s")
    f = pl.kernel(
        _body,
        out_type=jax.ShapeDtypeStruct((NC, B, D), jnp.float32),
        mesh=mesh,
        compiler_params=pltpu.CompilerParams(
            needs_layout_passes=False, use_tc_tiling_on_sc=False),
        scratch_types=[
            pltpu.VMEM((IDXC,), jnp.int32),
            pltpu.VMEM((NEC,), jnp.int32),
            pltpu.VMEM((NROW, D), jnp.float32),
            pltpu.VMEM((CB, D), jnp.float32),
            pltpu.VMEM_SHARED((HALF, D), jnp.float32),
            pltpu.SemaphoreType.DMA,
        ],
    )
    partial = f(x, mask_i, w)
    return pl.pallas_call(
        _combine_body,
        out_shape=jax.ShapeDtypeStruct((B, D), jnp.float32),
    )(partial)


def kernel(x, mask, W):
    return _run(x.reshape(B * S), mask.astype(jnp.int32).reshape(B * S), W)


# speedup vs baseline: 21.3625x; 1.3706x over previous
import functools

import jax
import jax.numpy as jnp
from jax import lax
from jax.experimental import pallas as pl
from jax.experimental.pallas import tpu as pltpu
from jax.experimental.pallas import tpu_sc as plsc

B = 4096
S = 200
D = 32
NUM_ROWS = 100000

NC = 2
NS = 16
HALF = NUM_ROWS // NC
RPT = B // NS
CB = 2
CHUNKS = RPT // CB
NEC = CB * S
NGRP = NEC // 16
IDXC = 528
NROW = 512


def _body(x_hbm, m_hbm, w_hbm, out_hbm,
          idx_r, idx_c, msk_v, rows_v, outst_v, shared_w,
          sem_in, sem_g, sem_out):
    c = lax.axis_index("c")
    sid = lax.axis_index("s")
    base = sid * RPT
    hbase = c * HALF

    @pl.when(sid == 0)
    def _stage():
        pltpu.sync_copy(w_hbm.at[pl.ds(hbase, HALF)], shared_w)

    plsc.subcore_barrier()

    inv_s = jnp.float32(1.0 / S)
    zero16i = jnp.zeros((16,), jnp.int32)
    zero16f = jnp.zeros((16,), jnp.float32)
    lo8 = lax.iota(jnp.int32, 16) < 8

    e00 = base * S
    pltpu.async_copy(x_hbm.at[pl.ds(e00, NEC)], idx_r.at[pl.ds(0, NEC)],
                     sem_in)
    pltpu.async_copy(m_hbm.at[pl.ds(e00, NEC)], msk_v.at[pl.ds(0, NEC)],
                     sem_in)

    def chunk_body(g, carry):
        row0 = base + g * CB
        e0 = row0 * S
        pltpu.make_async_copy(x_hbm.at[pl.ds(e0, NEC)],
                              idx_r.at[pl.ds(0, NEC)], sem_in).wait()
        pltpu.make_async_copy(m_hbm.at[pl.ds(e0, NEC)],
                              msk_v.at[pl.ds(0, NEC)], sem_in).wait()

        off = jnp.int32(0)
        bounds = [off]
        for grp in range(NGRP):
            e = grp * 16
            m16 = msk_v[pl.ds(e, 16)]
            i16 = idx_r[pl.ds(e, 16)] - hbase
            keep = (m16 != 0) & (i16 >= 0) & (i16 < HALF)
            if grp == 12:
                ka = keep & lo8
                plsc.store_compressed(idx_c.at[pl.ds(off, 16)], i16, mask=ka)
                off = off + plsc.all_reduce_population_count(ka)[0]
                bounds.append(off)
                kb = keep & (~lo8)
                plsc.store_compressed(idx_c.at[pl.ds(off, 16)], i16, mask=kb)
                off = off + plsc.all_reduce_population_count(kb)[0]
            else:
                plsc.store_compressed(idx_c.at[pl.ds(off, 16)], i16,
                                      mask=keep)
                off = off + plsc.all_reduce_population_count(keep)[0]
        bounds.append(off)

        @pl.when(g < CHUNKS - 1)
        def _prefetch():
            en = e0 + NEC
            pltpu.async_copy(x_hbm.at[pl.ds(en, NEC)],
                             idx_r.at[pl.ds(0, NEC)], sem_in)
            pltpu.async_copy(m_hbm.at[pl.ds(en, NEC)],
                             msk_v.at[pl.ds(0, NEC)], sem_in)

        for t in range(8):
            idx_c[pl.ds(off + t * 16, 16)] = zero16i

        ngrp = (off + 127) // 128

        def gbody(k, carry2):
            pltpu.async_copy(shared_w.at[idx_c.at[pl.ds(k * 128, 128)]],
                             rows_v.at[pl.ds(k * 128, 128)], sem_g).wait()
            return carry2

        lax.fori_loop(0, ngrp, gbody, 0)

        @pl.when(g > 0)
        def _drain():
            pltpu.make_async_copy(outst_v, out_hbm.at[c, pl.ds(row0, CB)],
                                  sem_out).wait()

        for j in range(CB):
            lo = bounds[j]
            hi = bounds[j + 1]
            n4 = lo + ((hi - lo) // 4) * 4

            def srow4(i, accs):
                a0, a1 = accs
                t = lo + i * 4
                a0 = (a0 + rows_v[t, pl.ds(0, 16)]
                      + rows_v[t + 1, pl.ds(0, 16)]
                      + rows_v[t + 2, pl.ds(0, 16)]
                      + rows_v[t + 3, pl.ds(0, 16)])
                a1 = (a1 + rows_v[t, pl.ds(16, 16)]
                      + rows_v[t + 1, pl.ds(16, 16)]
                      + rows_v[t + 2, pl.ds(16, 16)]
                      + rows_v[t + 3, pl.ds(16, 16)])
                return a0, a1

            def srow1(t, accs):
                a0, a1 = accs
                return (a0 + rows_v[t, pl.ds(0, 16)],
                        a1 + rows_v[t, pl.ds(16, 16)])

            accs = lax.fori_loop(0, (hi - lo) // 4, srow4,
                                 (zero16f, zero16f))
            a0, a1 = lax.fori_loop(n4, hi, srow1, accs)
            outst_v[j, pl.ds(0, 16)] = a0 * inv_s
            outst_v[j, pl.ds(16, 16)] = a1 * inv_s

        pltpu.async_copy(outst_v, out_hbm.at[c, pl.ds(row0, CB)], sem_out)
        return carry

    lax.fori_loop(0, CHUNKS, chunk_body, 0)
    pltpu.make_async_copy(outst_v, out_hbm.at[c, pl.ds(base, CB)],
                          sem_out).wait()


def _combine_body(p_ref, o_ref):
    o_ref[...] = p_ref[0] + p_ref[1]


@jax.jit
def _run(x, mask_i, w):
    mesh = plsc.VectorSubcoreMesh(core_axis_name="c", subcore_axis_name="s")
    f = pl.kernel(
        _body,
        out_type=jax.ShapeDtypeStruct((NC, B, D), jnp.float32),
        mesh=mesh,
        compiler_params=pltpu.CompilerParams(
            needs_layout_passes=False, use_tc_tiling_on_sc=False),
        scratch_types=[
            pltpu.VMEM((NEC,), jnp.int32),
            pltpu.VMEM((IDXC,), jnp.int32),
            pltpu.VMEM((NEC,), jnp.int32),
            pltpu.VMEM((NROW, D), jnp.float32),
            pltpu.VMEM((CB, D), jnp.float32),
            pltpu.VMEM_SHARED((HALF, D), jnp.float32),
            pltpu.SemaphoreType.DMA,
            pltpu.SemaphoreType.DMA,
            pltpu.SemaphoreType.DMA,
        ],
    )
    partial = f(x, mask_i, w)
    return pl.pallas_call(
        _combine_body,
        out_shape=jax.ShapeDtypeStruct((B, D), jnp.float32),
    )(partial)


def kernel(x, mask, W):
    return _run(x.reshape(B * S), mask.astype(jnp.int32).reshape(B * S), W)
